# Initial kernel scaffold; baseline (speedup 1.0000x reference)
import jax
import jax.numpy as jnp
from jax.experimental import pallas as pl


def kernel(x, edge_index, W1, phi1_w, b1, W2, phi2_w, b2, Wc, bc):
    raise NotImplementedError("write your pallas kernel here")


# speedup vs baseline: 9.0503x; 9.0503x over previous
import functools

import jax
import jax.numpy as jnp
from jax import lax
from jax.experimental import pallas as pl
from jax.experimental.pallas import tpu as pltpu
from jax.experimental.pallas import tpu_sc as plsc

N = 10000
E = 160000
D = 256
HALF = 128
EPS = 0.1
GAMMA = 0.1

NC = 2
NS = 16
RPT = 624
RPT_LAST = N - 15 * RPT
CH = 128
NCHUNK = E // CH
DEG_LANES = 16

_MESH = plsc.VectorSubcoreMesh(
    core_axis_name="c", subcore_axis_name="s", num_cores=NC, num_subcores=NS)

BN = 1000
GRID = N // BN


def _fill(ref, rows, value):
    cols = ref.shape[1] // 16

    def body(i, _):
        ref[i // cols, pl.ds((i % cols) * 16, 16)] = jnp.full((16,), value, jnp.float32)
        return 0

    lax.fori_loop(0, rows * cols, body, 0)


def _zero_acc(acc, zbuf, s):
    _fill(zbuf, 16, 0.0)
    nk = jnp.where(s == NS - 1, RPT_LAST // 16, RPT // 16)

    def body(k, _):
        pltpu.sync_copy(zbuf, acc.at[pl.ds(s * RPT + k * 16, 16)])
        return 0

    lax.fori_loop(0, nk, body, 0)


def _writeback(acc, out_hbm, s):
    @pl.when(s < NS - 1)
    def _():
        pltpu.sync_copy(acc.at[pl.ds(s * RPT, RPT)], out_hbm.at[pl.ds(s * RPT, RPT)])

    @pl.when(s == NS - 1)
    def _():
        sl = pl.ds(15 * RPT, RPT_LAST)
        pltpu.sync_copy(acc.at[sl], out_hbm.at[sl])


def _deg_body(dst_hbm, d0_hbm, d1_hbm, didx, ones, zbuf, acc):
    c = lax.axis_index("c")
    s = lax.axis_index("s")
    _fill(ones, CH, 1.0)
    _zero_acc(acc, zbuf, s)
    plsc.subcore_barrier()

    half = NCHUNK // NC

    def chunk(k, _):
        m = k * NS + s

        @pl.when(m < half)
        def _():
            pltpu.sync_copy(dst_hbm.at[pl.ds((c * half + m) * CH, CH)], didx)
            pltpu.sync_copy(ones, acc.at[didx], add=True)

        return 0

    lax.fori_loop(0, (half + NS - 1) // NS, chunk, 0)
    plsc.subcore_barrier()

    @pl.when(c == 0)
    def _():
        _writeback(acc, d0_hbm, s)

    @pl.when(c == 1)
    def _():
        _writeback(acc, d1_hbm, s)


_deg_call = pl.kernel(
    _deg_body,
    out_type=[jax.ShapeDtypeStruct((N, DEG_LANES), jnp.float32)] * 2,
    mesh=_MESH,
    scratch_types=[
        pltpu.VMEM((CH,), jnp.int32),
        pltpu.VMEM((CH, DEG_LANES), jnp.float32),
        pltpu.VMEM((16, DEG_LANES), jnp.float32),
        pltpu.VMEM_SHARED((N, DEG_LANES), jnp.float32),
    ],
)


def _seg_body(y0_hbm, y1_hbm, src_hbm, dst_hbm, s0_hbm, s1_hbm,
              sidx, didx, gbuf, zbuf, acc, sem):
    c = lax.axis_index("c")
    s = lax.axis_index("s")
    _zero_acc(acc, zbuf, s)
    plsc.subcore_barrier()

    def run(y_hbm):
        def chunk(k, _):
            m = k * NS + s

            @pl.when(m < NCHUNK)
            def _():
                off = m * CH
                pltpu.sync_copy(src_hbm.at[pl.ds(off, CH)], sidx)
                pltpu.sync_copy(dst_hbm.at[pl.ds(off, CH)], didx)
                pltpu.async_copy(y_hbm.at[sidx], gbuf, sem).wait()
                pltpu.sync_copy(gbuf, acc.at[didx], add=True)

            return 0

        lax.fori_loop(0, (NCHUNK + NS - 1) // NS, chunk, 0)

    @pl.when(c == 0)
    def _():
        run(y0_hbm)

    @pl.when(c == 1)
    def _():
        run(y1_hbm)

    plsc.subcore_barrier()

    @pl.when(c == 0)
    def _():
        _writeback(acc, s0_hbm, s)

    @pl.when(c == 1)
    def _():
        _writeback(acc, s1_hbm, s)


_seg_call = pl.kernel(
    _seg_body,
    out_type=[jax.ShapeDtypeStruct((N, HALF), jnp.float32)] * 2,
    mesh=_MESH,
    scratch_types=[
        pltpu.VMEM((CH,), jnp.int32),
        pltpu.VMEM((CH,), jnp.int32),
        pltpu.VMEM((CH, HALF), jnp.float32),
        pltpu.VMEM((16, HALF), jnp.float32),
        pltpu.VMEM_SHARED((N, HALF), jnp.float32),
        pltpu.SemaphoreType.DMA,
    ],
)


EPT = 5008

def _edge_body(pq_hbm, src_hbm, dst_hbm, out_hbm, pqv, sidxv, didxv, obuf):
    c = lax.axis_index("c")
    s = lax.axis_index("s")
    w = s * NC + c
    pltpu.sync_copy(pq_hbm, pqv)
    iota = lax.iota(jnp.int32, 16)

    def do(base_e, n_e, ngroups):
        pltpu.sync_copy(src_hbm.at[pl.ds(base_e, n_e)], sidxv.at[pl.ds(0, n_e)])
        pltpu.sync_copy(dst_hbm.at[pl.ds(base_e, n_e)], didxv.at[pl.ds(0, n_e)])

        def g(j, _):
            s4 = sidxv[pl.ds(j * 16, 16)] * 4
            d4 = didxv[pl.ds(j * 16, 16)] * 4
            p0 = plsc.load_gather(pqv, [s4])
            p1 = plsc.load_gather(pqv, [s4 + 1])
            q0 = plsc.load_gather(pqv, [d4 + 2])
            q1 = plsc.load_gather(pqv, [d4 + 3])
            r2 = (j * 16 + iota) * 2
            plsc.store_scatter(obuf, [r2], p0 + q0)
            plsc.store_scatter(obuf, [r2 + 1], p1 + q1)
            return 0

        lax.fori_loop(0, ngroups, g, 0)
        pltpu.sync_copy(obuf.at[pl.ds(0, 2 * n_e)],
                        out_hbm.at[pl.ds(2 * base_e, 2 * n_e)])

    @pl.when(w < 31)
    def _():
        do(w * EPT, EPT, EPT // 16)

    @pl.when(w == 31)
    def _():
        do(31 * EPT, E - 31 * EPT, (E - 31 * EPT) // 16)


_edge_call = pl.kernel(
    _edge_body,
    out_type=jax.ShapeDtypeStruct((2 * E,), jnp.float32),
    mesh=_MESH,
    scratch_types=[
        pltpu.VMEM((4 * N,), jnp.float32),
        pltpu.VMEM((EPT,), jnp.int32),
        pltpu.VMEM((EPT,), jnp.int32),
        pltpu.VMEM((2 * EPT,), jnp.float32),
    ],
    compiler_params=pltpu.CompilerParams(needs_layout_passes=False),
)


def _dis(d0_ref, d1_ref):
    return lax.rsqrt(d0_ref[:, 0:1] + d1_ref[:, 0:1] + 1.0)


def _mm1_body(x_ref, phiT_ref, aWT_ref, d0_ref, d1_ref, y0_ref, y1_ref, xa_ref):
    dis = _dis(d0_ref, d1_ref)
    y = jnp.dot(x_ref[...], phiT_ref[...], preferred_element_type=jnp.float32) * dis
    y0_ref[...] = y[:, :HALF]
    y1_ref[...] = y[:, HALF:]
    xa_ref[...] = jnp.dot(x_ref[...], aWT_ref[...], preferred_element_type=jnp.float32)


def _cmb1_body(x_ref, xa_ref, s0_ref, s1_ref, y0_ref, y1_ref, d0_ref, d1_ref,
               b_ref, phi2T_ref, aW2T_ref, h1_ref, y20_ref, y21_ref, xa2_ref):
    dis = _dis(d0_ref, d1_ref)
    agg = jnp.concatenate(
        [s0_ref[...] + y0_ref[...], s1_ref[...] + y1_ref[...]], axis=1) * dis
    h = jnp.tanh(xa_ref[...] + agg + b_ref[...])
    h1 = jnp.maximum(x_ref[...] + EPS * h, 0.0)
    h1_ref[...] = h1
    y2 = jnp.dot(h1, phi2T_ref[...], preferred_element_type=jnp.float32) * dis
    y20_ref[...] = y2[:, :HALF]
    y21_ref[...] = y2[:, HALF:]
    xa2_ref[...] = jnp.dot(h1, aW2T_ref[...], preferred_element_type=jnp.float32)


def _cmb2_body(h1_ref, xa2_ref, s0_ref, s1_ref, y0_ref, y1_ref, d0_ref, d1_ref,
               b_ref, wl_ref, wr_ref, bc_ref, pq_ref):
    dis = _dis(d0_ref, d1_ref)
    agg = jnp.concatenate(
        [s0_ref[...] + y0_ref[...], s1_ref[...] + y1_ref[...]], axis=1) * dis
    h2 = h1_ref[...] + EPS * jnp.tanh(xa2_ref[...] + agg + b_ref[...])
    p = jnp.dot(h2, wl_ref[...], preferred_element_type=jnp.float32) + bc_ref[...]
    q = jnp.dot(h2, wr_ref[...], preferred_element_type=jnp.float32)
    pq_ref[...] = jnp.concatenate([p, q], axis=1)


def _row_spec(cols):
    return pl.BlockSpec((BN, cols), lambda i: (i, 0))


def _full_spec(shape):
    return pl.BlockSpec(shape, lambda i: tuple(0 for _ in shape))


_f32 = jnp.float32

_mm1_call = pl.pallas_call(
    _mm1_body,
    grid=(GRID,),
    in_specs=[_row_spec(D), _full_spec((D, D)), _full_spec((D, D)),
              _row_spec(DEG_LANES), _row_spec(DEG_LANES)],
    out_specs=[_row_spec(HALF), _row_spec(HALF), _row_spec(D)],
    out_shape=[jax.ShapeDtypeStruct((N, HALF), _f32),
               jax.ShapeDtypeStruct((N, HALF), _f32),
               jax.ShapeDtypeStruct((N, D), _f32)],
)

_cmb1_call = pl.pallas_call(
    _cmb1_body,
    grid=(GRID,),
    in_specs=[_row_spec(D), _row_spec(D), _row_spec(HALF), _row_spec(HALF),
              _row_spec(HALF), _row_spec(HALF),
              _row_spec(DEG_LANES), _row_spec(DEG_LANES),
              _full_spec((1, D)), _full_spec((D, D)), _full_spec((D, D))],
    out_specs=[_row_spec(D), _row_spec(HALF), _row_spec(HALF), _row_spec(D)],
    out_shape=[jax.ShapeDtypeStruct((N, D), _f32),
               jax.ShapeDtypeStruct((N, HALF), _f32),
               jax.ShapeDtypeStruct((N, HALF), _f32),
               jax.ShapeDtypeStruct((N, D), _f32)],
)

_cmb2_call = pl.pallas_call(
    _cmb2_body,
    grid=(GRID,),
    in_specs=[_row_spec(D), _row_spec(D), _row_spec(HALF), _row_spec(HALF),
              _row_spec(HALF), _row_spec(HALF),
              _row_spec(DEG_LANES), _row_spec(DEG_LANES),
              _full_spec((1, D)), _full_spec((D, 2)), _full_spec((D, 2)),
              _full_spec((1, 2))],
    out_specs=[_row_spec(4)],
    out_shape=[jax.ShapeDtypeStruct((N, 4), _f32)],
)


def kernel(x, edge_index, W1, phi1_w, b1, W2, phi2_w, b2, Wc, bc):
    src = edge_index[0]
    dst = edge_index[1]
    eye = GAMMA * jnp.eye(D, dtype=jnp.float32)
    phi1T = phi1_w.T
    aW1T = W1.T - W1 - eye
    phi2T = phi2_w.T
    aW2T = W2.T - W2 - eye
    wl = Wc[:, :D].T
    wr = Wc[:, D:].T
    b1r = b1[None, :]
    b2r = b2[None, :]
    bcr = bc[None, :]

    d0, d1 = _deg_call(dst)
    y0, y1, xa1 = _mm1_call(x, phi1T, aW1T, d0, d1)
    s0, s1 = _seg_call(y0, y1, src, dst)
    h1, y20, y21, xa2 = _cmb1_call(x, xa1, s0, s1, y0, y1, d0, d1, b1r, phi2T, aW2T)
    s20, s21 = _seg_call(y20, y21, src, dst)
    (pq,) = _cmb2_call(h1, xa2, s20, s21, y20, y21, d0, d1, b2r, wl, wr, bcr)
    return _edge_call(pq.reshape(-1), src, dst).reshape(E, 2)


# speedup vs baseline: 12.1171x; 1.3389x over previous
import functools

import jax
import jax.numpy as jnp
from jax import lax
from jax.experimental import pallas as pl
from jax.experimental.pallas import tpu as pltpu
from jax.experimental.pallas import tpu_sc as plsc

N = 10000
E = 160000
D = 256
HALF = 128
EPS = 0.1
GAMMA = 0.1

NC = 2
NS = 16
RPT = 624
RPT_LAST = N - 15 * RPT
CH = 128
NCHUNK = E // CH
DEG_LANES = 16

_MESH = plsc.VectorSubcoreMesh(
    core_axis_name="c", subcore_axis_name="s", num_cores=NC, num_subcores=NS)

BN = 1000
GRID = N // BN


def _fill(ref, rows, value):
    cols = ref.shape[1] // 16

    def body(i, _):
        ref[i // cols, pl.ds((i % cols) * 16, 16)] = jnp.full((16,), value, jnp.float32)
        return 0

    lax.fori_loop(0, rows * cols, body, 0)


def _zero_acc(acc, zbuf, s):
    _fill(zbuf, 16, 0.0)
    nk = jnp.where(s == NS - 1, RPT_LAST // 16, RPT // 16)

    def body(k, _):
        pltpu.sync_copy(zbuf, acc.at[pl.ds(s * RPT + k * 16, 16)])
        return 0

    lax.fori_loop(0, nk, body, 0)


def _writeback(acc, out_hbm, s):
    @pl.when(s < NS - 1)
    def _():
        pltpu.sync_copy(acc.at[pl.ds(s * RPT, RPT)], out_hbm.at[pl.ds(s * RPT, RPT)])

    @pl.when(s == NS - 1)
    def _():
        sl = pl.ds(15 * RPT, RPT_LAST)
        pltpu.sync_copy(acc.at[sl], out_hbm.at[sl])


def _deg_body(dst_hbm, d0_hbm, d1_hbm, didx, ones, zbuf, acc):
    c = lax.axis_index("c")
    s = lax.axis_index("s")
    _fill(ones, CH, 1.0)
    _zero_acc(acc, zbuf, s)
    plsc.subcore_barrier()

    half = NCHUNK // NC

    def chunk(k, _):
        m = k * NS + s

        @pl.when(m < half)
        def _():
            pltpu.sync_copy(dst_hbm.at[pl.ds((c * half + m) * CH, CH)], didx)
            pltpu.sync_copy(ones, acc.at[didx], add=True)

        return 0

    lax.fori_loop(0, (half + NS - 1) // NS, chunk, 0)
    plsc.subcore_barrier()

    @pl.when(c == 0)
    def _():
        _writeback(acc, d0_hbm, s)

    @pl.when(c == 1)
    def _():
        _writeback(acc, d1_hbm, s)


_deg_call = pl.kernel(
    _deg_body,
    out_type=[jax.ShapeDtypeStruct((N, DEG_LANES), jnp.float32)] * 2,
    mesh=_MESH,
    scratch_types=[
        pltpu.VMEM((CH,), jnp.int32),
        pltpu.VMEM((CH, DEG_LANES), jnp.float32),
        pltpu.VMEM((16, DEG_LANES), jnp.float32),
        pltpu.VMEM_SHARED((N, DEG_LANES), jnp.float32),
    ],
)


CPT = (NCHUNK + NS - 1) // NS
CPT_LAST = NCHUNK - (NS - 1) * CPT


def _seg_body(y0_hbm, y1_hbm, src_hbm, dst_hbm, s0_hbm, s1_hbm,
              sidx0, sidx1, didx0, didx1, gbuf0, gbuf1, zbuf, acc, g0, g1):
    c = lax.axis_index("c")
    s = lax.axis_index("s")
    _zero_acc(acc, zbuf, s)
    plsc.subcore_barrier()

    base = s * CPT
    nch = jnp.where(s == NS - 1, CPT_LAST, CPT)

    def run(y_hbm):
        def prime(sidx, didx, gbuf, sem, m):
            off = (base + m) * CH
            pltpu.sync_copy(src_hbm.at[pl.ds(off, CH)], sidx)
            pltpu.sync_copy(dst_hbm.at[pl.ds(off, CH)], didx)
            pltpu.async_copy(y_hbm.at[sidx], gbuf, sem)

        def drain_scatter(sidx, didx, gbuf, sem):
            pltpu.make_async_copy(y_hbm.at[sidx], gbuf, sem).wait()
            pltpu.sync_copy(gbuf, acc.at[didx], add=True)

        prime(sidx0, didx0, gbuf0, g0, 0)

        def body(i, _):
            @pl.when(i % 2 == 0)
            def _():
                @pl.when(i + 1 < nch)
                def _():
                    prime(sidx1, didx1, gbuf1, g1, i + 1)
                drain_scatter(sidx0, didx0, gbuf0, g0)

            @pl.when(i % 2 == 1)
            def _():
                @pl.when(i + 1 < nch)
                def _():
                    prime(sidx0, didx0, gbuf0, g0, i + 1)
                drain_scatter(sidx1, didx1, gbuf1, g1)

            return 0

        lax.fori_loop(0, nch, body, 0)

    @pl.when(c == 0)
    def _():
        run(y0_hbm)

    @pl.when(c == 1)
    def _():
        run(y1_hbm)

    plsc.subcore_barrier()

    @pl.when(c == 0)
    def _():
        _writeback(acc, s0_hbm, s)

    @pl.when(c == 1)
    def _():
        _writeback(acc, s1_hbm, s)


_seg_call = pl.kernel(
    _seg_body,
    out_type=[jax.ShapeDtypeStruct((N, HALF), jnp.float32)] * 2,
    mesh=_MESH,
    scratch_types=[
        pltpu.VMEM((CH,), jnp.int32),
        pltpu.VMEM((CH,), jnp.int32),
        pltpu.VMEM((CH,), jnp.int32),
        pltpu.VMEM((CH,), jnp.int32),
        pltpu.VMEM((CH, HALF), jnp.float32),
        pltpu.VMEM((CH, HALF), jnp.float32),
        pltpu.VMEM((16, HALF), jnp.float32),
        pltpu.VMEM_SHARED((N, HALF), jnp.float32),
        pltpu.SemaphoreType.DMA,
        pltpu.SemaphoreType.DMA,
    ],
)


EPT = 5008

def _edge_body(pq_hbm, src_hbm, dst_hbm, out_hbm, pqv, sidxv, didxv, obuf):
    c = lax.axis_index("c")
    s = lax.axis_index("s")
    w = s * NC + c
    pltpu.sync_copy(pq_hbm, pqv)
    iota = lax.iota(jnp.int32, 16)

    def do(base_e, n_e, ngroups):
        pltpu.sync_copy(src_hbm.at[pl.ds(base_e, n_e)], sidxv.at[pl.ds(0, n_e)])
        pltpu.sync_copy(dst_hbm.at[pl.ds(base_e, n_e)], didxv.at[pl.ds(0, n_e)])

        def g(j, _):
            s4 = sidxv[pl.ds(j * 16, 16)] * 4
            d4 = didxv[pl.ds(j * 16, 16)] * 4
            p0 = plsc.load_gather(pqv, [s4])
            p1 = plsc.load_gather(pqv, [s4 + 1])
            q0 = plsc.load_gather(pqv, [d4 + 2])
            q1 = plsc.load_gather(pqv, [d4 + 3])
            r2 = (j * 16 + iota) * 2
            plsc.store_scatter(obuf, [r2], p0 + q0)
            plsc.store_scatter(obuf, [r2 + 1], p1 + q1)
            return 0

        lax.fori_loop(0, ngroups, g, 0)
        pltpu.sync_copy(obuf.at[pl.ds(0, 2 * n_e)],
                        out_hbm.at[pl.ds(2 * base_e, 2 * n_e)])

    @pl.when(w < 31)
    def _():
        do(w * EPT, EPT, EPT // 16)

    @pl.when(w == 31)
    def _():
        do(31 * EPT, E - 31 * EPT, (E - 31 * EPT) // 16)


_edge_call = pl.kernel(
    _edge_body,
    out_type=jax.ShapeDtypeStruct((2 * E,), jnp.float32),
    mesh=_MESH,
    scratch_types=[
        pltpu.VMEM((4 * N,), jnp.float32),
        pltpu.VMEM((EPT,), jnp.int32),
        pltpu.VMEM((EPT,), jnp.int32),
        pltpu.VMEM((2 * EPT,), jnp.float32),
    ],
    compiler_params=pltpu.CompilerParams(needs_layout_passes=False),
)


def _dis(d0_ref, d1_ref):
    return lax.rsqrt(d0_ref[:, 0:1] + d1_ref[:, 0:1] + 1.0)


def _mm1_body(x_ref, phiT_ref, aWT_ref, d0_ref, d1_ref, y0_ref, y1_ref, xa_ref):
    dis = _dis(d0_ref, d1_ref)
    y = jnp.dot(x_ref[...], phiT_ref[...], preferred_element_type=jnp.float32) * dis
    y0_ref[...] = y[:, :HALF]
    y1_ref[...] = y[:, HALF:]
    xa_ref[...] = jnp.dot(x_ref[...], aWT_ref[...], preferred_element_type=jnp.float32)


def _cmb1_body(x_ref, xa_ref, s0_ref, s1_ref, y0_ref, y1_ref, d0_ref, d1_ref,
               b_ref, phi2T_ref, aW2T_ref, h1_ref, y20_ref, y21_ref, xa2_ref):
    dis = _dis(d0_ref, d1_ref)
    agg = jnp.concatenate(
        [s0_ref[...] + y0_ref[...], s1_ref[...] + y1_ref[...]], axis=1) * dis
    h = jnp.tanh(xa_ref[...] + agg + b_ref[...])
    h1 = jnp.maximum(x_ref[...] + EPS * h, 0.0)
    h1_ref[...] = h1
    y2 = jnp.dot(h1, phi2T_ref[...], preferred_element_type=jnp.float32) * dis
    y20_ref[...] = y2[:, :HALF]
    y21_ref[...] = y2[:, HALF:]
    xa2_ref[...] = jnp.dot(h1, aW2T_ref[...], preferred_element_type=jnp.float32)


def _cmb2_body(h1_ref, xa2_ref, s0_ref, s1_ref, y0_ref, y1_ref, d0_ref, d1_ref,
               b_ref, wl_ref, wr_ref, bc_ref, pq_ref):
    dis = _dis(d0_ref, d1_ref)
    agg = jnp.concatenate(
        [s0_ref[...] + y0_ref[...], s1_ref[...] + y1_ref[...]], axis=1) * dis
    h2 = h1_ref[...] + EPS * jnp.tanh(xa2_ref[...] + agg + b_ref[...])
    p = jnp.dot(h2, wl_ref[...], preferred_element_type=jnp.float32) + bc_ref[...]
    q = jnp.dot(h2, wr_ref[...], preferred_element_type=jnp.float32)
    pq_ref[...] = jnp.concatenate([p, q], axis=1)


def _row_spec(cols):
    return pl.BlockSpec((BN, cols), lambda i: (i, 0))


def _full_spec(shape):
    return pl.BlockSpec(shape, lambda i: tuple(0 for _ in shape))


_f32 = jnp.float32

_mm1_call = pl.pallas_call(
    _mm1_body,
    grid=(GRID,),
    in_specs=[_row_spec(D), _full_spec((D, D)), _full_spec((D, D)),
              _row_spec(DEG_LANES), _row_spec(DEG_LANES)],
    out_specs=[_row_spec(HALF), _row_spec(HALF), _row_spec(D)],
    out_shape=[jax.ShapeDtypeStruct((N, HALF), _f32),
               jax.ShapeDtypeStruct((N, HALF), _f32),
               jax.ShapeDtypeStruct((N, D), _f32)],
)

_cmb1_call = pl.pallas_call(
    _cmb1_body,
    grid=(GRID,),
    in_specs=[_row_spec(D), _row_spec(D), _row_spec(HALF), _row_spec(HALF),
              _row_spec(HALF), _row_spec(HALF),
              _row_spec(DEG_LANES), _row_spec(DEG_LANES),
              _full_spec((1, D)), _full_spec((D, D)), _full_spec((D, D))],
    out_specs=[_row_spec(D), _row_spec(HALF), _row_spec(HALF), _row_spec(D)],
    out_shape=[jax.ShapeDtypeStruct((N, D), _f32),
               jax.ShapeDtypeStruct((N, HALF), _f32),
               jax.ShapeDtypeStruct((N, HALF), _f32),
               jax.ShapeDtypeStruct((N, D), _f32)],
)

_cmb2_call = pl.pallas_call(
    _cmb2_body,
    grid=(GRID,),
    in_specs=[_row_spec(D), _row_spec(D), _row_spec(HALF), _row_spec(HALF),
              _row_spec(HALF), _row_spec(HALF),
              _row_spec(DEG_LANES), _row_spec(DEG_LANES),
              _full_spec((1, D)), _full_spec((D, 2)), _full_spec((D, 2)),
              _full_spec((1, 2))],
    out_specs=[_row_spec(4)],
    out_shape=[jax.ShapeDtypeStruct((N, 4), _f32)],
)


def kernel(x, edge_index, W1, phi1_w, b1, W2, phi2_w, b2, Wc, bc):
    src = edge_index[0]
    dst = edge_index[1]
    eye = GAMMA * jnp.eye(D, dtype=jnp.float32)
    phi1T = phi1_w.T
    aW1T = W1.T - W1 - eye
    phi2T = phi2_w.T
    aW2T = W2.T - W2 - eye
    wl = Wc[:, :D].T
    wr = Wc[:, D:].T
    b1r = b1[None, :]
    b2r = b2[None, :]
    bcr = bc[None, :]

    d0, d1 = _deg_call(dst)
    y0, y1, xa1 = _mm1_call(x, phi1T, aW1T, d0, d1)
    s0, s1 = _seg_call(y0, y1, src, dst)
    h1, y20, y21, xa2 = _cmb1_call(x, xa1, s0, s1, y0, y1, d0, d1, b1r, phi2T, aW2T)
    s20, s21 = _seg_call(y20, y21, src, dst)
    (pq,) = _cmb2_call(h1, xa2, s20, s21, y20, y21, d0, d1, b2r, wl, wr, bcr)
    return _edge_call(pq.reshape(-1), src, dst).reshape(E, 2)


# speedup vs baseline: 12.7799x; 1.0547x over previous
import functools

import jax
import jax.numpy as jnp
from jax import lax
from jax.experimental import pallas as pl
from jax.experimental.pallas import tpu as pltpu
from jax.experimental.pallas import tpu_sc as plsc

N = 10000
E = 160000
D = 256
HALF = 128
EPS = 0.1
GAMMA = 0.1

NC = 2
NS = 16
RPT = 624
RPT_LAST = N - 15 * RPT
CH = 128
NCHUNK = E // CH
NCHUNK_P = 1256
N_PAD = N + 8
DEG_LANES = 16

_MESH = plsc.VectorSubcoreMesh(
    core_axis_name="c", subcore_axis_name="s", num_cores=NC, num_subcores=NS)

BN = 1000
GRID = N // BN


def _fill(ref, rows, value):
    cols = ref.shape[1] // 16

    def body(i, _):
        ref[i // cols, pl.ds((i % cols) * 16, 16)] = jnp.full((16,), value, jnp.float32)
        return 0

    lax.fori_loop(0, rows * cols, body, 0)


def _zero_acc(acc, zbuf, s):
    _fill(zbuf, 16, 0.0)
    nk = jnp.where(s == NS - 1, RPT_LAST // 16, RPT // 16)

    def body(k, _):
        pltpu.sync_copy(zbuf, acc.at[pl.ds(s * RPT + k * 16, 16)])
        return 0

    lax.fori_loop(0, nk, body, 0)


def _writeback(acc, out_hbm, s):
    @pl.when(s < NS - 1)
    def _():
        pltpu.sync_copy(acc.at[pl.ds(s * RPT, RPT)], out_hbm.at[pl.ds(s * RPT, RPT)])

    @pl.when(s == NS - 1)
    def _():
        sl = pl.ds(15 * RPT, RPT_LAST)
        pltpu.sync_copy(acc.at[sl], out_hbm.at[sl])


DCPW = 40
DCPW_LAST = NCHUNK_P - 31 * DCPW


def _deg_body(dst_hbm, d0_hbm, d1_hbm, didx, ones, zbuf, acc):
    c = lax.axis_index("c")
    s = lax.axis_index("s")
    w = s * NC + c
    _fill(ones, CH, 1.0)
    _zero_acc(acc, zbuf, s)
    base = w * DCPW
    nch = jnp.where(w == 31, DCPW_LAST, DCPW)

    @pl.when(w < 31)
    def _():
        pltpu.sync_copy(dst_hbm.at[pl.ds(base, DCPW)], didx)

    @pl.when(w == 31)
    def _():
        pltpu.sync_copy(dst_hbm.at[pl.ds(base, DCPW_LAST)],
                        didx.at[pl.ds(0, DCPW_LAST)])

    plsc.subcore_barrier()

    def chunk(k, _):
        pltpu.sync_copy(ones, acc.at[didx.at[k]], add=True)
        return 0

    lax.fori_loop(0, nch, chunk, 0)
    plsc.subcore_barrier()

    @pl.when(c == 0)
    def _():
        _writeback(acc, d0_hbm, s)

    @pl.when(c == 1)
    def _():
        _writeback(acc, d1_hbm, s)


_deg_call = pl.kernel(
    _deg_body,
    out_type=[jax.ShapeDtypeStruct((N, DEG_LANES), jnp.float32)] * 2,
    mesh=_MESH,
    scratch_types=[
        pltpu.VMEM((DCPW, CH), jnp.int32),
        pltpu.VMEM((CH, DEG_LANES), jnp.float32),
        pltpu.VMEM((16, DEG_LANES), jnp.float32),
        pltpu.VMEM_SHARED((N_PAD, DEG_LANES), jnp.float32),
    ],
)


CPT = 80
CPT_LAST = NCHUNK_P - (NS - 1) * CPT


SLAB = 8


def _seg_body(y0_hbm, y1_hbm, src_hbm, dst_hbm, s0_hbm, s1_hbm,
              sidx0, sidx1, didx0, didx1, gbuf0, gbuf1, zbuf, acc,
              g0, g1, is0, is1, id0, id1):
    c = lax.axis_index("c")
    s = lax.axis_index("s")
    _zero_acc(acc, zbuf, s)

    base = s * CPT
    nch = jnp.where(s == NS - 1, CPT_LAST, CPT)
    nslab = nch // SLAB

    pltpu.sync_copy(src_hbm.at[pl.ds(base, SLAB)], sidx0)
    pltpu.sync_copy(dst_hbm.at[pl.ds(base, SLAB)], didx0)
    plsc.subcore_barrier()

    def slab_refs(sl):
        return (src_hbm.at[pl.ds(base + sl * SLAB, SLAB)],
                dst_hbm.at[pl.ds(base + sl * SLAB, SLAB)])

    def slab_copy(sl):
        sref, dref = slab_refs(sl)

        @pl.when(sl % 2 == 0)
        def _():
            pltpu.async_copy(sref, sidx0, is0)
            pltpu.async_copy(dref, didx0, id0)

        @pl.when(sl % 2 == 1)
        def _():
            pltpu.async_copy(sref, sidx1, is1)
            pltpu.async_copy(dref, didx1, id1)

    def slab_wait(sl):
        sref, dref = slab_refs(sl)

        @pl.when(sl % 2 == 0)
        def _():
            pltpu.make_async_copy(sref, sidx0, is0).wait()
            pltpu.make_async_copy(dref, didx0, id0).wait()

        @pl.when(sl % 2 == 1)
        def _():
            pltpu.make_async_copy(sref, sidx1, is1).wait()
            pltpu.make_async_copy(dref, didx1, id1).wait()

    def run(y_hbm):
        def gather_start(i, gbuf, gsem):
            w = i % SLAB

            @pl.when((i // SLAB) % 2 == 0)
            def _():
                pltpu.async_copy(y_hbm.at[sidx0.at[w]], gbuf, gsem)

            @pl.when((i // SLAB) % 2 == 1)
            def _():
                pltpu.async_copy(y_hbm.at[sidx1.at[w]], gbuf, gsem)

        def gather_wait(i, gbuf, gsem):
            w = i % SLAB

            @pl.when((i // SLAB) % 2 == 0)
            def _():
                pltpu.make_async_copy(y_hbm.at[sidx0.at[w]], gbuf, gsem).wait()

            @pl.when((i // SLAB) % 2 == 1)
            def _():
                pltpu.make_async_copy(y_hbm.at[sidx1.at[w]], gbuf, gsem).wait()

        def scatter(i, gbuf):
            w = i % SLAB

            @pl.when((i // SLAB) % 2 == 0)
            def _():
                pltpu.sync_copy(gbuf, acc.at[didx0.at[w]], add=True)

            @pl.when((i // SLAB) % 2 == 1)
            def _():
                pltpu.sync_copy(gbuf, acc.at[didx1.at[w]], add=True)

        pltpu.async_copy(y_hbm.at[sidx0.at[0]], gbuf0, g0)

        def body(i, _):
            @pl.when(((i + 1) % SLAB == 0) & (i + 1 < nch))
            def _():
                slab_wait((i + 1) // SLAB)

            @pl.when((i % SLAB == 1) & (i // SLAB + 1 < nslab))
            def _():
                slab_copy(i // SLAB + 1)

            @pl.when(i % 2 == 0)
            def _():
                @pl.when(i + 1 < nch)
                def _():
                    gather_start(i + 1, gbuf1, g1)
                gather_wait(i, gbuf0, g0)
                scatter(i, gbuf0)

            @pl.when(i % 2 == 1)
            def _():
                @pl.when(i + 1 < nch)
                def _():
                    gather_start(i + 1, gbuf0, g0)
                gather_wait(i, gbuf1, g1)
                scatter(i, gbuf1)

            return 0

        lax.fori_loop(0, nch, body, 0)

    @pl.when(c == 0)
    def _():
        run(y0_hbm)

    @pl.when(c == 1)
    def _():
        run(y1_hbm)

    plsc.subcore_barrier()

    @pl.when(c == 0)
    def _():
        _writeback(acc, s0_hbm, s)

    @pl.when(c == 1)
    def _():
        _writeback(acc, s1_hbm, s)


_seg_call = pl.kernel(
    _seg_body,
    out_type=[jax.ShapeDtypeStruct((N, HALF), jnp.float32)] * 2,
    mesh=_MESH,
    scratch_types=[
        pltpu.VMEM((SLAB, CH), jnp.int32),
        pltpu.VMEM((SLAB, CH), jnp.int32),
        pltpu.VMEM((SLAB, CH), jnp.int32),
        pltpu.VMEM((SLAB, CH), jnp.int32),
        pltpu.VMEM((CH, HALF), jnp.float32),
        pltpu.VMEM((CH, HALF), jnp.float32),
        pltpu.VMEM((16, HALF), jnp.float32),
        pltpu.VMEM_SHARED((N_PAD, HALF), jnp.float32),
        pltpu.SemaphoreType.DMA,
        pltpu.SemaphoreType.DMA,
        pltpu.SemaphoreType.DMA,
        pltpu.SemaphoreType.DMA,
        pltpu.SemaphoreType.DMA,
        pltpu.SemaphoreType.DMA,
    ],
)


EPT = 5008

def _edge_body(pq_hbm, src_hbm, dst_hbm, out_hbm, pqv, sidxv, didxv, obuf):
    c = lax.axis_index("c")
    s = lax.axis_index("s")
    w = s * NC + c
    pltpu.sync_copy(pq_hbm, pqv)
    iota = lax.iota(jnp.int32, 16)

    def do(base_e, n_e, ngroups):
        pltpu.sync_copy(src_hbm.at[pl.ds(base_e, n_e)], sidxv.at[pl.ds(0, n_e)])
        pltpu.sync_copy(dst_hbm.at[pl.ds(base_e, n_e)], didxv.at[pl.ds(0, n_e)])

        def g(j, _):
            s4 = sidxv[pl.ds(j * 16, 16)] * 4
            d4 = didxv[pl.ds(j * 16, 16)] * 4
            p0 = plsc.load_gather(pqv, [s4])
            p1 = plsc.load_gather(pqv, [s4 + 1])
            q0 = plsc.load_gather(pqv, [d4 + 2])
            q1 = plsc.load_gather(pqv, [d4 + 3])
            r2 = (j * 16 + iota) * 2
            plsc.store_scatter(obuf, [r2], p0 + q0)
            plsc.store_scatter(obuf, [r2 + 1], p1 + q1)
            return 0

        lax.fori_loop(0, ngroups, g, 0)
        pltpu.sync_copy(obuf.at[pl.ds(0, 2 * n_e)],
                        out_hbm.at[pl.ds(2 * base_e, 2 * n_e)])

    @pl.when(w < 31)
    def _():
        do(w * EPT, EPT, EPT // 16)

    @pl.when(w == 31)
    def _():
        do(31 * EPT, E - 31 * EPT, (E - 31 * EPT) // 16)


_edge_call = pl.kernel(
    _edge_body,
    out_type=jax.ShapeDtypeStruct((2 * E,), jnp.float32),
    mesh=_MESH,
    scratch_types=[
        pltpu.VMEM((4 * N,), jnp.float32),
        pltpu.VMEM((EPT,), jnp.int32),
        pltpu.VMEM((EPT,), jnp.int32),
        pltpu.VMEM((2 * EPT,), jnp.float32),
    ],
    compiler_params=pltpu.CompilerParams(needs_layout_passes=False),
)


def _dis(d0_ref, d1_ref):
    return lax.rsqrt(d0_ref[:, 0:1] + d1_ref[:, 0:1] + 1.0)


def _mm1_body(x_ref, phiT_ref, aWT_ref, d0_ref, d1_ref, y0_ref, y1_ref, xa_ref):
    dis = _dis(d0_ref, d1_ref)
    y = jnp.dot(x_ref[...], phiT_ref[...], preferred_element_type=jnp.float32) * dis
    y0_ref[...] = y[:, :HALF]
    y1_ref[...] = y[:, HALF:]
    xa_ref[...] = jnp.dot(x_ref[...], aWT_ref[...], preferred_element_type=jnp.float32)


def _cmb1_body(x_ref, xa_ref, s0_ref, s1_ref, y0_ref, y1_ref, d0_ref, d1_ref,
               b_ref, phi2T_ref, aW2T_ref, h1_ref, y20_ref, y21_ref, xa2_ref):
    dis = _dis(d0_ref, d1_ref)
    agg = jnp.concatenate(
        [s0_ref[...] + y0_ref[...], s1_ref[...] + y1_ref[...]], axis=1) * dis
    h = jnp.tanh(xa_ref[...] + agg + b_ref[...])
    h1 = jnp.maximum(x_ref[...] + EPS * h, 0.0)
    h1_ref[...] = h1
    y2 = jnp.dot(h1, phi2T_ref[...], preferred_element_type=jnp.float32) * dis
    y20_ref[...] = y2[:, :HALF]
    y21_ref[...] = y2[:, HALF:]
    xa2_ref[...] = jnp.dot(h1, aW2T_ref[...], preferred_element_type=jnp.float32)


def _cmb2_body(h1_ref, xa2_ref, s0_ref, s1_ref, y0_ref, y1_ref, d0_ref, d1_ref,
               b_ref, wl_ref, wr_ref, bc_ref, pq_ref):
    dis = _dis(d0_ref, d1_ref)
    agg = jnp.concatenate(
        [s0_ref[...] + y0_ref[...], s1_ref[...] + y1_ref[...]], axis=1) * dis
    h2 = h1_ref[...] + EPS * jnp.tanh(xa2_ref[...] + agg + b_ref[...])
    p = jnp.dot(h2, wl_ref[...], preferred_element_type=jnp.float32) + bc_ref[...]
    q = jnp.dot(h2, wr_ref[...], preferred_element_type=jnp.float32)
    pq_ref[...] = jnp.concatenate([p, q], axis=1)


def _row_spec(cols):
    return pl.BlockSpec((BN, cols), lambda i: (i, 0))


def _full_spec(shape):
    return pl.BlockSpec(shape, lambda i: tuple(0 for _ in shape))


_f32 = jnp.float32

_mm1_call = pl.pallas_call(
    _mm1_body,
    grid=(GRID,),
    in_specs=[_row_spec(D), _full_spec((D, D)), _full_spec((D, D)),
              _row_spec(DEG_LANES), _row_spec(DEG_LANES)],
    out_specs=[_row_spec(HALF), _row_spec(HALF), _row_spec(D)],
    out_shape=[jax.ShapeDtypeStruct((N, HALF), _f32),
               jax.ShapeDtypeStruct((N, HALF), _f32),
               jax.ShapeDtypeStruct((N, D), _f32)],
)

_cmb1_call = pl.pallas_call(
    _cmb1_body,
    grid=(GRID,),
    in_specs=[_row_spec(D), _row_spec(D), _row_spec(HALF), _row_spec(HALF),
              _row_spec(HALF), _row_spec(HALF),
              _row_spec(DEG_LANES), _row_spec(DEG_LANES),
              _full_spec((1, D)), _full_spec((D, D)), _full_spec((D, D))],
    out_specs=[_row_spec(D), _row_spec(HALF), _row_spec(HALF), _row_spec(D)],
    out_shape=[jax.ShapeDtypeStruct((N, D), _f32),
               jax.ShapeDtypeStruct((N, HALF), _f32),
               jax.ShapeDtypeStruct((N, HALF), _f32),
               jax.ShapeDtypeStruct((N, D), _f32)],
)

_cmb2_call = pl.pallas_call(
    _cmb2_body,
    grid=(GRID,),
    in_specs=[_row_spec(D), _row_spec(D), _row_spec(HALF), _row_spec(HALF),
              _row_spec(HALF), _row_spec(HALF),
              _row_spec(DEG_LANES), _row_spec(DEG_LANES),
              _full_spec((1, D)), _full_spec((D, 2)), _full_spec((D, 2)),
              _full_spec((1, 2))],
    out_specs=[_row_spec(4)],
    out_shape=[jax.ShapeDtypeStruct((N, 4), _f32)],
)


def kernel(x, edge_index, W1, phi1_w, b1, W2, phi2_w, b2, Wc, bc):
    src = edge_index[0]
    dst = edge_index[1]
    eye = GAMMA * jnp.eye(D, dtype=jnp.float32)
    phi1T = phi1_w.T
    aW1T = W1.T - W1 - eye
    phi2T = phi2_w.T
    aW2T = W2.T - W2 - eye
    wl = Wc[:, :D].T
    wr = Wc[:, D:].T
    b1r = b1[None, :]
    b2r = b2[None, :]
    bcr = bc[None, :]

    pad = NCHUNK_P - NCHUNK
    src2 = jnp.pad(src.reshape(NCHUNK, CH), ((0, pad), (0, 0)))
    dst2 = jnp.pad(dst.reshape(NCHUNK, CH), ((0, pad), (0, 0)),
                   constant_values=N)
    d0, d1 = _deg_call(dst2)
    y0, y1, xa1 = _mm1_call(x, phi1T, aW1T, d0, d1)
    s0, s1 = _seg_call(y0, y1, src2, dst2)
    h1, y20, y21, xa2 = _cmb1_call(x, xa1, s0, s1, y0, y1, d0, d1, b1r, phi2T, aW2T)
    s20, s21 = _seg_call(y20, y21, src2, dst2)
    (pq,) = _cmb2_call(h1, xa2, s20, s21, y20, y21, d0, d1, b2r, wl, wr, bcr)
    return _edge_call(pq.reshape(-1), src, dst).reshape(E, 2)


# speedup vs baseline: 13.0815x; 1.0236x over previous
import functools

import jax
import jax.numpy as jnp
from jax import lax
from jax.experimental import pallas as pl
from jax.experimental.pallas import tpu as pltpu
from jax.experimental.pallas import tpu_sc as plsc

N = 10000
E = 160000
D = 256
HALF = 128
EPS = 0.1
GAMMA = 0.1

NC = 2
NS = 16
RPT = 624
RPT_LAST = N - 15 * RPT
CH = 128
NCHUNK = E // CH
NCHUNK_P = 1256
N_PAD = N + 8
DEG_LANES = 16

_MESH = plsc.VectorSubcoreMesh(
    core_axis_name="c", subcore_axis_name="s", num_cores=NC, num_subcores=NS)

BN = 1000
GRID = N // BN


def _fill(ref, rows, value):
    cols = ref.shape[1] // 16

    def body(i, _):
        ref[i // cols, pl.ds((i % cols) * 16, 16)] = jnp.full((16,), value, jnp.float32)
        return 0

    lax.fori_loop(0, rows * cols, body, 0)


def _zero_acc(acc, zbuf, s):
    _fill(zbuf, 16, 0.0)
    nk = jnp.where(s == NS - 1, RPT_LAST // 16, RPT // 16)

    def body(k, _):
        pltpu.sync_copy(zbuf, acc.at[pl.ds(s * RPT + k * 16, 16)])
        return 0

    lax.fori_loop(0, nk, body, 0)


def _writeback(acc, out_hbm, s):
    @pl.when(s < NS - 1)
    def _():
        pltpu.sync_copy(acc.at[pl.ds(s * RPT, RPT)], out_hbm.at[pl.ds(s * RPT, RPT)])

    @pl.when(s == NS - 1)
    def _():
        sl = pl.ds(15 * RPT, RPT_LAST)
        pltpu.sync_copy(acc.at[sl], out_hbm.at[sl])


DCPW = 40
DCPW_LAST = NCHUNK_P - 31 * DCPW


def _deg_body(dst_hbm, d0_hbm, d1_hbm, didx, ones, zbuf, acc):
    c = lax.axis_index("c")
    s = lax.axis_index("s")
    w = s * NC + c
    _fill(ones, CH, 1.0)
    _zero_acc(acc, zbuf, s)
    base = w * DCPW
    nch = jnp.where(w == 31, DCPW_LAST, DCPW)

    @pl.when(w < 31)
    def _():
        pltpu.sync_copy(dst_hbm.at[pl.ds(base, DCPW)], didx)

    @pl.when(w == 31)
    def _():
        pltpu.sync_copy(dst_hbm.at[pl.ds(base, DCPW_LAST)],
                        didx.at[pl.ds(0, DCPW_LAST)])

    plsc.subcore_barrier()

    def chunk(k, _):
        pltpu.sync_copy(ones, acc.at[didx.at[k]], add=True)
        return 0

    lax.fori_loop(0, nch, chunk, 0)
    plsc.subcore_barrier()

    @pl.when(c == 0)
    def _():
        _writeback(acc, d0_hbm, s)

    @pl.when(c == 1)
    def _():
        _writeback(acc, d1_hbm, s)


_deg_call = pl.kernel(
    _deg_body,
    out_type=[jax.ShapeDtypeStruct((N, DEG_LANES), jnp.float32)] * 2,
    mesh=_MESH,
    scratch_types=[
        pltpu.VMEM((DCPW, CH), jnp.int32),
        pltpu.VMEM((CH, DEG_LANES), jnp.float32),
        pltpu.VMEM((16, DEG_LANES), jnp.float32),
        pltpu.VMEM_SHARED((N_PAD, DEG_LANES), jnp.float32),
    ],
)


CPT = 80
CPT_LAST = NCHUNK_P - (NS - 1) * CPT


SLAB = 8


def _seg_body(y0_hbm, y1_hbm, src_hbm, dst_hbm, s0_hbm, s1_hbm,
              sidx0, sidx1, didx0, didx1, gbuf0, gbuf1, zbuf, acc,
              g0, g1, is0, is1, id0, id1):
    c = lax.axis_index("c")
    s = lax.axis_index("s")
    _zero_acc(acc, zbuf, s)

    base = s * CPT
    nch = jnp.where(s == NS - 1, CPT_LAST, CPT)
    nslab = nch // SLAB

    pltpu.sync_copy(src_hbm.at[pl.ds(base, SLAB)], sidx0)
    pltpu.sync_copy(dst_hbm.at[pl.ds(base, SLAB)], didx0)
    plsc.subcore_barrier()

    def slab_refs(sl):
        return (src_hbm.at[pl.ds(base + sl * SLAB, SLAB)],
                dst_hbm.at[pl.ds(base + sl * SLAB, SLAB)])

    def slab_copy(sl):
        sref, dref = slab_refs(sl)

        @pl.when(sl % 2 == 0)
        def _():
            pltpu.async_copy(sref, sidx0, is0)
            pltpu.async_copy(dref, didx0, id0)

        @pl.when(sl % 2 == 1)
        def _():
            pltpu.async_copy(sref, sidx1, is1)
            pltpu.async_copy(dref, didx1, id1)

    def slab_wait(sl):
        sref, dref = slab_refs(sl)

        @pl.when(sl % 2 == 0)
        def _():
            pltpu.make_async_copy(sref, sidx0, is0).wait()
            pltpu.make_async_copy(dref, didx0, id0).wait()

        @pl.when(sl % 2 == 1)
        def _():
            pltpu.make_async_copy(sref, sidx1, is1).wait()
            pltpu.make_async_copy(dref, didx1, id1).wait()

    def run(y_hbm):
        def gather_start(i, gbuf, gsem):
            w = i % SLAB

            @pl.when((i // SLAB) % 2 == 0)
            def _():
                pltpu.async_copy(y_hbm.at[sidx0.at[w]], gbuf, gsem)

            @pl.when((i // SLAB) % 2 == 1)
            def _():
                pltpu.async_copy(y_hbm.at[sidx1.at[w]], gbuf, gsem)

        def gather_wait(i, gbuf, gsem):
            w = i % SLAB

            @pl.when((i // SLAB) % 2 == 0)
            def _():
                pltpu.make_async_copy(y_hbm.at[sidx0.at[w]], gbuf, gsem).wait()

            @pl.when((i // SLAB) % 2 == 1)
            def _():
                pltpu.make_async_copy(y_hbm.at[sidx1.at[w]], gbuf, gsem).wait()

        def scatter(i, gbuf):
            w = i % SLAB

            @pl.when((i // SLAB) % 2 == 0)
            def _():
                pltpu.sync_copy(gbuf, acc.at[didx0.at[w]], add=True)

            @pl.when((i // SLAB) % 2 == 1)
            def _():
                pltpu.sync_copy(gbuf, acc.at[didx1.at[w]], add=True)

        pltpu.async_copy(y_hbm.at[sidx0.at[0]], gbuf0, g0)

        def body(i, _):
            @pl.when(((i + 1) % SLAB == 0) & (i + 1 < nch))
            def _():
                slab_wait((i + 1) // SLAB)

            @pl.when((i % SLAB == 1) & (i // SLAB + 1 < nslab))
            def _():
                slab_copy(i // SLAB + 1)

            @pl.when(i % 2 == 0)
            def _():
                @pl.when(i + 1 < nch)
                def _():
                    gather_start(i + 1, gbuf1, g1)
                gather_wait(i, gbuf0, g0)
                scatter(i, gbuf0)

            @pl.when(i % 2 == 1)
            def _():
                @pl.when(i + 1 < nch)
                def _():
                    gather_start(i + 1, gbuf0, g0)
                gather_wait(i, gbuf1, g1)
                scatter(i, gbuf1)

            return 0

        lax.fori_loop(0, nch, body, 0)

    @pl.when(c == 0)
    def _():
        run(y0_hbm)

    @pl.when(c == 1)
    def _():
        run(y1_hbm)

    plsc.subcore_barrier()

    @pl.when(c == 0)
    def _():
        _writeback(acc, s0_hbm, s)

    @pl.when(c == 1)
    def _():
        _writeback(acc, s1_hbm, s)


_seg_call = pl.kernel(
    _seg_body,
    out_type=[jax.ShapeDtypeStruct((N, HALF), jnp.float32)] * 2,
    mesh=_MESH,
    scratch_types=[
        pltpu.VMEM((SLAB, CH), jnp.int32),
        pltpu.VMEM((SLAB, CH), jnp.int32),
        pltpu.VMEM((SLAB, CH), jnp.int32),
        pltpu.VMEM((SLAB, CH), jnp.int32),
        pltpu.VMEM((CH, HALF), jnp.float32),
        pltpu.VMEM((CH, HALF), jnp.float32),
        pltpu.VMEM((16, HALF), jnp.float32),
        pltpu.VMEM_SHARED((N_PAD, HALF), jnp.float32),
        pltpu.SemaphoreType.DMA,
        pltpu.SemaphoreType.DMA,
        pltpu.SemaphoreType.DMA,
        pltpu.SemaphoreType.DMA,
        pltpu.SemaphoreType.DMA,
        pltpu.SemaphoreType.DMA,
    ],
)


EPT = 5008

def _edge_body(pq_hbm, src_hbm, dst_hbm, out_hbm, pqv, sidxv, didxv, obuf):
    c = lax.axis_index("c")
    s = lax.axis_index("s")
    w = s * NC + c
    pltpu.sync_copy(pq_hbm, pqv)
    iota = lax.iota(jnp.int32, 16)

    def do(base_e, n_e, ngroups):
        pltpu.sync_copy(src_hbm.at[pl.ds(base_e, n_e)], sidxv.at[pl.ds(0, n_e)])
        pltpu.sync_copy(dst_hbm.at[pl.ds(base_e, n_e)], didxv.at[pl.ds(0, n_e)])

        def g(j, _):
            s4 = sidxv[pl.ds(j * 16, 16)] * 4
            d4 = didxv[pl.ds(j * 16, 16)] * 4
            p0 = plsc.load_gather(pqv, [s4])
            p1 = plsc.load_gather(pqv, [s4 + 1])
            q0 = plsc.load_gather(pqv, [d4 + 2])
            q1 = plsc.load_gather(pqv, [d4 + 3])
            r2 = (j * 16 + iota) * 2
            plsc.store_scatter(obuf, [r2], p0 + q0)
            plsc.store_scatter(obuf, [r2 + 1], p1 + q1)
            return 0

        lax.fori_loop(0, ngroups, g, 0)
        pltpu.sync_copy(obuf.at[pl.ds(0, 2 * n_e)],
                        out_hbm.at[pl.ds(2 * base_e, 2 * n_e)])

    @pl.when(w < 31)
    def _():
        do(w * EPT, EPT, EPT // 16)

    @pl.when(w == 31)
    def _():
        do(31 * EPT, E - 31 * EPT, (E - 31 * EPT) // 16)


_edge_call = pl.kernel(
    _edge_body,
    out_type=jax.ShapeDtypeStruct((2 * E,), jnp.float32),
    mesh=_MESH,
    scratch_types=[
        pltpu.VMEM((4 * N,), jnp.float32),
        pltpu.VMEM((EPT,), jnp.int32),
        pltpu.VMEM((EPT,), jnp.int32),
        pltpu.VMEM((2 * EPT,), jnp.float32),
    ],
    compiler_params=pltpu.CompilerParams(needs_layout_passes=False),
)


def _dis(d0_ref, d1_ref):
    return lax.rsqrt(d0_ref[:, 0:1] + d1_ref[:, 0:1] + 1.0)


def _mm1_body(x_ref, phiT_ref, d0_ref, d1_ref, y0_ref, y1_ref):
    dis = _dis(d0_ref, d1_ref)
    y = jnp.dot(x_ref[...], phiT_ref[...], preferred_element_type=jnp.float32) * dis
    y0_ref[...] = y[:, :HALF]
    y1_ref[...] = y[:, HALF:]


def _cmb1_body(x_ref, s0_ref, s1_ref, y0_ref, y1_ref, d0_ref, d1_ref,
               b_ref, aWT_ref, phi2T_ref, h1_ref, y20_ref, y21_ref):
    dis = _dis(d0_ref, d1_ref)
    agg = jnp.concatenate(
        [s0_ref[...] + y0_ref[...], s1_ref[...] + y1_ref[...]], axis=1) * dis
    xa = jnp.dot(x_ref[...], aWT_ref[...], preferred_element_type=jnp.float32)
    h = jnp.tanh(xa + agg + b_ref[...])
    h1 = jnp.maximum(x_ref[...] + EPS * h, 0.0)
    h1_ref[...] = h1
    y2 = jnp.dot(h1, phi2T_ref[...], preferred_element_type=jnp.float32) * dis
    y20_ref[...] = y2[:, :HALF]
    y21_ref[...] = y2[:, HALF:]


def _cmb2_body(h1_ref, s0_ref, s1_ref, y0_ref, y1_ref, d0_ref, d1_ref,
               b_ref, aWT_ref, wl_ref, wr_ref, bc_ref, pq_ref):
    dis = _dis(d0_ref, d1_ref)
    agg = jnp.concatenate(
        [s0_ref[...] + y0_ref[...], s1_ref[...] + y1_ref[...]], axis=1) * dis
    xa2 = jnp.dot(h1_ref[...], aWT_ref[...], preferred_element_type=jnp.float32)
    h2 = h1_ref[...] + EPS * jnp.tanh(xa2 + agg + b_ref[...])
    p = jnp.dot(h2, wl_ref[...], preferred_element_type=jnp.float32) + bc_ref[...]
    q = jnp.dot(h2, wr_ref[...], preferred_element_type=jnp.float32)
    pq_ref[...] = jnp.concatenate([p, q], axis=1)


def _row_spec(cols):
    return pl.BlockSpec((BN, cols), lambda i: (i, 0))


def _full_spec(shape):
    return pl.BlockSpec(shape, lambda i: tuple(0 for _ in shape))


_f32 = jnp.float32

_mm1_call = pl.pallas_call(
    _mm1_body,
    grid=(GRID,),
    in_specs=[_row_spec(D), _full_spec((D, D)),
              _row_spec(DEG_LANES), _row_spec(DEG_LANES)],
    out_specs=[_row_spec(HALF), _row_spec(HALF)],
    out_shape=[jax.ShapeDtypeStruct((N, HALF), _f32),
               jax.ShapeDtypeStruct((N, HALF), _f32)],
)

_cmb1_call = pl.pallas_call(
    _cmb1_body,
    grid=(GRID,),
    in_specs=[_row_spec(D), _row_spec(HALF), _row_spec(HALF),
              _row_spec(HALF), _row_spec(HALF),
              _row_spec(DEG_LANES), _row_spec(DEG_LANES),
              _full_spec((1, D)), _full_spec((D, D)), _full_spec((D, D))],
    out_specs=[_row_spec(D), _row_spec(HALF), _row_spec(HALF)],
    out_shape=[jax.ShapeDtypeStruct((N, D), _f32),
               jax.ShapeDtypeStruct((N, HALF), _f32),
               jax.ShapeDtypeStruct((N, HALF), _f32)],
)

_cmb2_call = pl.pallas_call(
    _cmb2_body,
    grid=(GRID,),
    in_specs=[_row_spec(D), _row_spec(HALF), _row_spec(HALF),
              _row_spec(HALF), _row_spec(HALF),
              _row_spec(DEG_LANES), _row_spec(DEG_LANES),
              _full_spec((1, D)), _full_spec((D, D)), _full_spec((D, 2)),
              _full_spec((D, 2)), _full_spec((1, 2))],
    out_specs=[_row_spec(4)],
    out_shape=[jax.ShapeDtypeStruct((N, 4), _f32)],
)


def kernel(x, edge_index, W1, phi1_w, b1, W2, phi2_w, b2, Wc, bc):
    src = edge_index[0]
    dst = edge_index[1]
    eye = GAMMA * jnp.eye(D, dtype=jnp.float32)
    phi1T = phi1_w.T
    aW1T = W1.T - W1 - eye
    phi2T = phi2_w.T
    aW2T = W2.T - W2 - eye
    wl = Wc[:, :D].T
    wr = Wc[:, D:].T
    b1r = b1[None, :]
    b2r = b2[None, :]
    bcr = bc[None, :]

    pad = NCHUNK_P - NCHUNK
    src2 = jnp.pad(src.reshape(NCHUNK, CH), ((0, pad), (0, 0)))
    dst2 = jnp.pad(dst.reshape(NCHUNK, CH), ((0, pad), (0, 0)),
                   constant_values=N)
    d0, d1 = _deg_call(dst2)
    y0, y1 = _mm1_call(x, phi1T, d0, d1)
    s0, s1 = _seg_call(y0, y1, src2, dst2)
    h1, y20, y21 = _cmb1_call(x, s0, s1, y0, y1, d0, d1, b1r, aW1T, phi2T)
    s20, s21 = _seg_call(y20, y21, src2, dst2)
    (pq,) = _cmb2_call(h1, s20, s21, y20, y21, d0, d1, b2r, aW2T, wl, wr, bcr)
    return _edge_call(pq.reshape(-1), src, dst).reshape(E, 2)
